# baseline (speedup 1.0000x reference)
import jax
import jax.numpy as jnp
from jax import lax
from jax.experimental import pallas as pl
from jax.experimental.pallas import tpu as pltpu

_BS = 32
_DX = 512
_DY = 512


def _tc_body(xl_ref, yl_ref, hist_ref, pc_ref, w_ref):
    b = pl.program_id(0)
    x0 = xl_ref[b, 0]
    y0 = yl_ref[b, 0]
    wx = (xl_ref[b, 1] - x0) * (1.0 / _DX)
    wy = (yl_ref[b, 1] - y0) * (1.0 / _DY)
    bx = x0 + wx * 0.5
    by = y0 + wy * 0.5

    rows = lax.broadcasted_iota(jnp.int32, (_DX, 2 * _DY), 0)
    cols = lax.broadcasted_iota(jnp.int32, (_DX, 2 * _DY), 1)
    vy = (cols >> 1).astype(jnp.float32) * wy + by
    vx = rows.astype(jnp.float32) * wx + bx
    pc_ref[0] = jnp.where((cols & 1) == 0, vx, vy)
    w_ref[0] = hist_ref[0]


def kernel(hist, x_lims, y_lims):
    pc3, w3 = pl.pallas_call(
        _tc_body,
        grid=(_BS,),
        in_specs=[
            pl.BlockSpec(memory_space=pltpu.SMEM),
            pl.BlockSpec(memory_space=pltpu.SMEM),
            pl.BlockSpec((1, _DX, _DY), lambda b: (b, 0, 0)),
        ],
        out_specs=[
            pl.BlockSpec((1, _DX, 2 * _DY), lambda b: (b, 0, 0)),
            pl.BlockSpec((1, _DX, _DY), lambda b: (b, 0, 0)),
        ],
        out_shape=[
            jax.ShapeDtypeStruct((_BS, _DX, 2 * _DY), jnp.float32),
            jax.ShapeDtypeStruct((_BS, _DX, _DY), jnp.float32),
        ],
    )(x_lims, y_lims, hist)
    return pc3.reshape(_BS, _DX * _DY, 2), w3.reshape(_BS, _DX * _DY)


# speedup vs baseline: 3.8506x; 3.8506x over previous
import jax
import jax.numpy as jnp
from jax import lax
from jax.experimental import pallas as pl
from jax.experimental.pallas import tpu as pltpu

_BS = 32
_DX = 512
_DY = 512
_NPT = _DX * _DY
_NCH = _NPT // 128
_NR = 2 * _NCH
_RB = 1024


def _pc_body(xl_ref, yl_ref, a_ref):
    b = pl.program_id(0)
    r0 = pl.program_id(1) * _RB
    x0 = xl_ref[b, 0]
    y0 = yl_ref[b, 0]
    wx = (xl_ref[b, 1] - x0) * (1.0 / _DX)
    wy = (yl_ref[b, 1] - y0) * (1.0 / _DY)
    bx = x0 + wx * 0.5
    by = y0 + wy * 0.5

    rows = r0 + lax.broadcasted_iota(jnp.int32, (_RB, 128), 0)
    lanes = lax.broadcasted_iota(jnp.int32, (_RB, 128), 1)
    k = ((rows >> 1) << 7) + lanes
    vx = (k >> 9).astype(jnp.float32) * wx + bx
    vy = (k & (_DY - 1)).astype(jnp.float32) * wy + by
    a_ref[0] = jnp.where((rows & 1) == 0, vx, vy)


def kernel(hist, x_lims, y_lims):
    a = pl.pallas_call(
        _pc_body,
        grid=(_BS, _NR // _RB),
        in_specs=[
            pl.BlockSpec(memory_space=pltpu.SMEM),
            pl.BlockSpec(memory_space=pltpu.SMEM),
        ],
        out_specs=pl.BlockSpec((1, _RB, 128), lambda b, r: (b, r, 0)),
        out_shape=jax.ShapeDtypeStruct((_BS, _NR, 128), jnp.float32),
    )(x_lims, y_lims)
    pc = a.reshape(_BS, _NCH, 2, 128).transpose(0, 1, 3, 2).reshape(_BS, _NPT, 2)
    return pc, hist.reshape(_BS, _NPT)


# speedup vs baseline: 4.1996x; 1.0906x over previous
import jax
import jax.numpy as jnp
from jax import lax
from jax.experimental import pallas as pl
from jax.experimental.pallas import tpu as pltpu

_BS = 32
_DX = 512
_DY = 512
_NPT = _DX * _DY
_NCH = _NPT // 128
_NR = 2 * _NCH
_RB = 1024


def _pc_body(xl_ref, yl_ref, a_ref):
    b = pl.program_id(0)
    r0 = pl.program_id(1) * _RB
    x0 = xl_ref[b, 0]
    y0 = yl_ref[b, 0]
    wx = (xl_ref[b, 1] - x0) * (1.0 / _DX)
    wy = (yl_ref[b, 1] - y0) * (1.0 / _DY)
    bx = x0 + wx * 0.5
    by = y0 + wy * 0.5

    sub = lax.broadcasted_iota(jnp.int32, (8, 128), 0)
    lane = lax.broadcasted_iota(jnp.int32, (8, 128), 1)
    ypat = (((sub >> 1) & 3) << 7) + lane
    y8 = ypat.astype(jnp.float32) * wy + by
    m8 = (sub & 1) == 0
    ng = _RB // 8
    yfull = jnp.broadcast_to(y8[None], (ng, 8, 128)).reshape(_RB, 128)
    mfull = jnp.broadcast_to(m8[None], (ng, 8, 128)).reshape(_RB, 128)

    q = (r0 + lax.broadcasted_iota(jnp.int32, (_RB, 1), 0)) >> 3
    xcol = q.astype(jnp.float32) * wx + bx
    a_ref[0] = jnp.where(mfull, jnp.broadcast_to(xcol, (_RB, 128)), yfull)


def kernel(hist, x_lims, y_lims):
    a = pl.pallas_call(
        _pc_body,
        grid=(_BS, _NR // _RB),
        in_specs=[
            pl.BlockSpec(memory_space=pltpu.SMEM),
            pl.BlockSpec(memory_space=pltpu.SMEM),
        ],
        out_specs=pl.BlockSpec((1, _RB, 128), lambda b, r: (b, r, 0)),
        out_shape=jax.ShapeDtypeStruct((_BS, _NR, 128), jnp.float32),
    )(x_lims, y_lims)
    pc = a.reshape(_BS, _NCH, 2, 128).transpose(0, 1, 3, 2).reshape(_BS, _NPT, 2)
    return pc, hist.reshape(_BS, _NPT)


# speedup vs baseline: 5.2152x; 1.2418x over previous
import jax
import jax.numpy as jnp
from jax import lax
from jax.experimental import pallas as pl
from jax.experimental.pallas import tpu as pltpu

_BS = 32
_DX = 512
_DY = 512
_NPT = _DX * _DY
_NCH = _NPT // 128
_NR = 2 * _NCH
_RB = 2048


def _pc_body(xl_ref, yl_ref, a_ref):
    b = pl.program_id(0)
    r0 = pl.program_id(1) * _RB
    x0 = xl_ref[b, 0]
    y0 = yl_ref[b, 0]
    wx = (xl_ref[b, 1] - x0) * (1.0 / _DX)
    wy = (yl_ref[b, 1] - y0) * (1.0 / _DY)
    bx = x0 + wx * 0.5
    by = y0 + wy * 0.5

    sub = lax.broadcasted_iota(jnp.int32, (8, 128), 0)
    lane = lax.broadcasted_iota(jnp.int32, (8, 128), 1)
    ypat = (((sub >> 1) & 3) << 7) + lane
    y8 = ypat.astype(jnp.float32) * wy + by
    m8 = (sub & 1) == 0
    ng = _RB // 8
    yfull = jnp.broadcast_to(y8[None], (ng, 8, 128)).reshape(_RB, 128)
    mfull = jnp.broadcast_to(m8[None], (ng, 8, 128)).reshape(_RB, 128)

    q = (r0 + lax.broadcasted_iota(jnp.int32, (_RB, 1), 0)) >> 3
    xcol = q.astype(jnp.float32) * wx + bx
    a_ref[0] = jnp.where(mfull, jnp.broadcast_to(xcol, (_RB, 128)), yfull)


def kernel(hist, x_lims, y_lims):
    a = pl.pallas_call(
        _pc_body,
        grid=(_BS, _NR // _RB),
        in_specs=[
            pl.BlockSpec(memory_space=pltpu.SMEM),
            pl.BlockSpec(memory_space=pltpu.SMEM),
        ],
        out_specs=pl.BlockSpec((1, _RB, 128), lambda b, r: (b, r, 0)),
        out_shape=jax.ShapeDtypeStruct((_BS, _NR, 128), jnp.float32),
    )(x_lims, y_lims)
    pc = a.reshape(_BS, _NCH, 2, 128).transpose(0, 1, 3, 2).reshape(_BS, _NPT, 2)
    return pc, hist.reshape(_BS, _NPT)


# speedup vs baseline: 5.7995x; 1.1120x over previous
import jax
import jax.numpy as jnp
from jax import lax
from jax.experimental import pallas as pl
from jax.experimental.pallas import tpu as pltpu

_BS = 32
_DX = 512
_DY = 512
_NPT = _DX * _DY
_NCH = _NPT // 128
_NR = 2 * _NCH
_RB = 4096


def _pc_body(xl_ref, yl_ref, a_ref):
    b = pl.program_id(0)
    r0 = pl.program_id(1) * _RB
    x0 = xl_ref[b, 0]
    y0 = yl_ref[b, 0]
    wx = (xl_ref[b, 1] - x0) * (1.0 / _DX)
    wy = (yl_ref[b, 1] - y0) * (1.0 / _DY)
    bx = x0 + wx * 0.5
    by = y0 + wy * 0.5

    sub = lax.broadcasted_iota(jnp.int32, (8, 128), 0)
    lane = lax.broadcasted_iota(jnp.int32, (8, 128), 1)
    ypat = (((sub >> 1) & 3) << 7) + lane
    y8 = ypat.astype(jnp.float32) * wy + by
    m8 = (sub & 1) == 0
    ng = _RB // 8
    yfull = jnp.broadcast_to(y8[None], (ng, 8, 128)).reshape(_RB, 128)
    mfull = jnp.broadcast_to(m8[None], (ng, 8, 128)).reshape(_RB, 128)

    q = (r0 + lax.broadcasted_iota(jnp.int32, (_RB, 1), 0)) >> 3
    xcol = q.astype(jnp.float32) * wx + bx
    a_ref[0] = jnp.where(mfull, jnp.broadcast_to(xcol, (_RB, 128)), yfull)


def kernel(hist, x_lims, y_lims):
    a = pl.pallas_call(
        _pc_body,
        grid=(_BS, _NR // _RB),
        in_specs=[
            pl.BlockSpec(memory_space=pltpu.SMEM),
            pl.BlockSpec(memory_space=pltpu.SMEM),
        ],
        out_specs=pl.BlockSpec((1, _RB, 128), lambda b, r: (b, r, 0)),
        out_shape=jax.ShapeDtypeStruct((_BS, _NR, 128), jnp.float32),
    )(x_lims, y_lims)
    pc = a.reshape(_BS, _NCH, 2, 128).transpose(0, 1, 3, 2).reshape(_BS, _NPT, 2)
    return pc, hist.reshape(_BS, _NPT)


# speedup vs baseline: 6.1132x; 1.0541x over previous
import jax
import jax.numpy as jnp
from jax import lax
from jax.experimental import pallas as pl
from jax.experimental.pallas import tpu as pltpu

_BS = 32
_DX = 512
_DY = 512
_NPT = _DX * _DY
_NCH = _NPT // 128
_NR = 2 * _NCH
_RB = 4096
_BB = 2


def _pc_body(xl_ref, yl_ref, a_ref):
    sub = lax.broadcasted_iota(jnp.int32, (8, 128), 0)
    lane = lax.broadcasted_iota(jnp.int32, (8, 128), 1)
    ypat = (((sub >> 1) & 3) << 7) + lane
    ypat_f = ypat.astype(jnp.float32)
    m8 = (sub & 1) == 0
    ng = _RB // 8
    mfull = jnp.broadcast_to(m8[None], (ng, 8, 128)).reshape(_RB, 128)
    q = lax.broadcasted_iota(jnp.int32, (_RB, 1), 0) >> 3
    qf = q.astype(jnp.float32)

    for i in range(_BB):
        b = pl.program_id(0) * _BB + i
        x0 = xl_ref[b, 0]
        y0 = yl_ref[b, 0]
        wx = (xl_ref[b, 1] - x0) * (1.0 / _DX)
        wy = (yl_ref[b, 1] - y0) * (1.0 / _DY)
        bx = x0 + wx * 0.5
        by = y0 + wy * 0.5
        y8 = ypat_f * wy + by
        yfull = jnp.broadcast_to(y8[None], (ng, 8, 128)).reshape(_RB, 128)
        xcol = qf * wx + bx
        a_ref[i] = jnp.where(mfull, jnp.broadcast_to(xcol, (_RB, 128)), yfull)


def kernel(hist, x_lims, y_lims):
    a = pl.pallas_call(
        _pc_body,
        grid=(_BS // _BB,),
        in_specs=[
            pl.BlockSpec(memory_space=pltpu.SMEM),
            pl.BlockSpec(memory_space=pltpu.SMEM),
        ],
        out_specs=pl.BlockSpec((_BB, _RB, 128), lambda b: (b, 0, 0)),
        out_shape=jax.ShapeDtypeStruct((_BS, _NR, 128), jnp.float32),
    )(x_lims, y_lims)
    pc = a.reshape(_BS, _NCH, 2, 128).transpose(0, 1, 3, 2).reshape(_BS, _NPT, 2)
    return pc, hist.reshape(_BS, _NPT)


# speedup vs baseline: 6.1586x; 1.0074x over previous
import jax
import jax.numpy as jnp
from jax import lax
from jax.experimental import pallas as pl
from jax.experimental.pallas import tpu as pltpu

_BS = 32
_DX = 512
_DY = 512
_NPT = _DX * _DY
_NCH = _NPT // 128
_NR = 2 * _NCH
_RB = 4096
_BB = 4


def _pc_body(xl_ref, yl_ref, a_ref):
    sub = lax.broadcasted_iota(jnp.int32, (8, 128), 0)
    lane = lax.broadcasted_iota(jnp.int32, (8, 128), 1)
    ypat = (((sub >> 1) & 3) << 7) + lane
    ypat_f = ypat.astype(jnp.float32)
    m8 = (sub & 1) == 0
    ng = _RB // 8
    mfull = jnp.broadcast_to(m8[None], (ng, 8, 128)).reshape(_RB, 128)
    q = lax.broadcasted_iota(jnp.int32, (_RB, 1), 0) >> 3
    qf = q.astype(jnp.float32)

    for i in range(_BB):
        b = pl.program_id(0) * _BB + i
        x0 = xl_ref[b, 0]
        y0 = yl_ref[b, 0]
        wx = (xl_ref[b, 1] - x0) * (1.0 / _DX)
        wy = (yl_ref[b, 1] - y0) * (1.0 / _DY)
        bx = x0 + wx * 0.5
        by = y0 + wy * 0.5
        y8 = ypat_f * wy + by
        yfull = jnp.broadcast_to(y8[None], (ng, 8, 128)).reshape(_RB, 128)
        xcol = qf * wx + bx
        a_ref[i] = jnp.where(mfull, jnp.broadcast_to(xcol, (_RB, 128)), yfull)


def kernel(hist, x_lims, y_lims):
    a = pl.pallas_call(
        _pc_body,
        grid=(_BS // _BB,),
        in_specs=[
            pl.BlockSpec(memory_space=pltpu.SMEM),
            pl.BlockSpec(memory_space=pltpu.SMEM),
        ],
        out_specs=pl.BlockSpec((_BB, _RB, 128), lambda b: (b, 0, 0)),
        out_shape=jax.ShapeDtypeStruct((_BS, _NR, 128), jnp.float32),
    )(x_lims, y_lims)
    pc = a.reshape(_BS, _NCH, 2, 128).transpose(0, 1, 3, 2).reshape(_BS, _NPT, 2)
    return pc, hist.reshape(_BS, _NPT)


# speedup vs baseline: 8.5157x; 1.3827x over previous
import jax
import jax.numpy as jnp
from jax import lax
from jax.experimental import pallas as pl
from jax.experimental.pallas import tpu as pltpu

_BS = 32
_DX = 512
_DY = 512
_NPT = _DX * _DY
_NCH = _NPT // 128
_NR = 2 * _NCH
_GB = 8
_RH = 2
_RB = _NR // _RH
_XR = _DX // _RH


def _body(xl_ref, yl_ref, hist_ref, a_ref, w_ref):
    rh = pl.program_id(1)
    sub = lax.broadcasted_iota(jnp.int32, (8, 128), 0)
    lane = lax.broadcasted_iota(jnp.int32, (8, 128), 1)
    ypat = (((sub >> 1) & 3) << 7) + lane
    ypat_f = ypat.astype(jnp.float32)
    m8 = (sub & 1) == 0
    ng = _RB // 8
    mfull = jnp.broadcast_to(m8[None], (ng, 8, 128)).reshape(_RB, 128)
    q = rh * _XR + (lax.broadcasted_iota(jnp.int32, (_RB, 1), 0) >> 3)
    qf = q.astype(jnp.float32)

    for i in range(_GB):
        b = pl.program_id(0) * _GB + i
        x0 = xl_ref[b, 0]
        y0 = yl_ref[b, 0]
        wx = (xl_ref[b, 1] - x0) * (1.0 / _DX)
        wy = (yl_ref[b, 1] - y0) * (1.0 / _DY)
        bx = x0 + wx * 0.5
        by = y0 + wy * 0.5
        y8 = ypat_f * wy + by
        yfull = jnp.broadcast_to(y8[None], (ng, 8, 128)).reshape(_RB, 128)
        xcol = qf * wx + bx
        a_ref[i] = jnp.where(mfull, jnp.broadcast_to(xcol, (_RB, 128)), yfull)

    w_ref[...] = hist_ref[...].reshape(_GB, _XR * _DY)


def kernel(hist, x_lims, y_lims):
    a, w = pl.pallas_call(
        _body,
        grid=(_BS // _GB, _RH),
        in_specs=[
            pl.BlockSpec(memory_space=pltpu.SMEM),
            pl.BlockSpec(memory_space=pltpu.SMEM),
            pl.BlockSpec((_GB, _XR, _DY), lambda g, r: (g, r, 0)),
        ],
        out_specs=[
            pl.BlockSpec((_GB, _RB, 128), lambda g, r: (g, r, 0)),
            pl.BlockSpec((_GB, _XR * _DY), lambda g, r: (g, r)),
        ],
        out_shape=[
            jax.ShapeDtypeStruct((_BS, _NR, 128), jnp.float32),
            jax.ShapeDtypeStruct((_BS, _NPT), jnp.float32),
        ],
    )(x_lims, y_lims, hist)
    pc = a.reshape(_BS, _NCH, 2, 128).transpose(0, 1, 3, 2).reshape(_BS, _NPT, 2)
    return pc, w
